# Initial kernel scaffold; baseline (speedup 1.0000x reference)
import jax
import jax.numpy as jnp
from jax.experimental import pallas as pl


def kernel(x, edge_index, W0, b0, Wc, W1, b1):
    raise NotImplementedError("write your pallas kernel here")


# speedup vs baseline: 5.1807x; 5.1807x over previous
import functools
import math

import jax
import jax.numpy as jnp
import numpy as np
from jax import lax
from jax.experimental import pallas as pl
from jax.experimental.pallas import tpu as pltpu
from jax.experimental.pallas import tpu_sc as plsc

ALPHA = 0.1
THETA = 0.5
CHUNK = 128


def _sc_info():
    try:
        info = plsc.get_sparse_core_info()
        return info.num_cores, info.num_subcores
    except Exception:
        return 2, 16


@functools.lru_cache(maxsize=None)
def _make_segment_sum(N, E, D):
    NC, NS = _sc_info()
    NW = NC * NS
    n_chunks = E // CHUNK
    iters = math.ceil(n_chunks / NW)
    n_full = N // CHUNK
    rem = N - n_full * CHUNK
    row_blocks = n_full + (1 if rem else 0)
    row_iters = math.ceil(row_blocks / NS)
    mesh = plsc.VectorSubcoreMesh(core_axis_name="c", subcore_axis_name="s")

    @functools.partial(
        pl.kernel,
        mesh=mesh,
        out_type=jax.ShapeDtypeStruct((NC, N, D), jnp.float32),
        scratch_types=[
            pltpu.VMEM((CHUNK,), jnp.int32),
            pltpu.VMEM((CHUNK,), jnp.int32),
            pltpu.VMEM((CHUNK, D), jnp.float32),
            pltpu.VMEM_SHARED((N, D), jnp.float32),
            pltpu.SemaphoreType.DMA,
        ],
    )
    def seg(h_hbm, src_hbm, dst_hbm, zeros_hbm, out_hbm, src_v, dst_v, rows_v, acc, sem):
        c = lax.axis_index("c")
        s = lax.axis_index("s")
        w = s * NC + c

        def for_each_row_block(fn):
            for i in range(row_iters):
                b = s + NS * i

                @pl.when(b < n_full)
                def _():
                    fn(b * CHUNK, CHUNK)

                if rem:

                    @pl.when(b == n_full)
                    def _():
                        fn(n_full * CHUNK, rem)

        for_each_row_block(lambda base, sz: pltpu.sync_copy(
            zeros_hbm.at[pl.ds(0, sz)], acc.at[pl.ds(base, sz)]))
        plsc.subcore_barrier()

        def body(i, carry):
            cw = w + NW * i

            @pl.when(cw < n_chunks)
            def _():
                base = cw * CHUNK
                pltpu.sync_copy(src_hbm.at[pl.ds(base, CHUNK)], src_v)
                pltpu.sync_copy(dst_hbm.at[pl.ds(base, CHUNK)], dst_v)
                pltpu.async_copy(h_hbm.at[src_v], rows_v, sem).wait()
                pltpu.sync_copy(rows_v, acc.at[dst_v], add=True)

            return carry

        lax.fori_loop(0, iters, body, None)
        plsc.subcore_barrier()
        for_each_row_block(lambda base, sz: pltpu.sync_copy(
            acc.at[pl.ds(base, sz)], out_hbm.at[c, pl.ds(base, sz)]))

    return seg


def _mm_relu_body(x_ref, w_ref, b_ref, o_ref):
    y = jnp.dot(x_ref[...], w_ref[...], preferred_element_type=jnp.float32)
    o_ref[...] = jnp.maximum(y + b_ref[...], 0.0)


def _combine_body(p0_ref, p1_ref, x0_ref, w_ref, o_ref, *, beta):
    z = (1.0 - ALPHA) * (p0_ref[...] + p1_ref[...]) + ALPHA * x0_ref[...]
    y = (1.0 - beta) * z + beta * jnp.dot(z, w_ref[...], preferred_element_type=jnp.float32)
    o_ref[...] = jnp.maximum(y, 0.0)


def _final_body(h_ref, w_ref, b_ref, o_ref, *, C):
    logits = jnp.dot(h_ref[...], w_ref[...], preferred_element_type=jnp.float32) + b_ref[...]
    col = lax.broadcasted_iota(jnp.int32, logits.shape, 1)
    valid = col < C
    masked = jnp.where(valid, logits, -jnp.inf)
    m = jnp.max(masked, axis=1, keepdims=True)
    ex = jnp.where(valid, jnp.exp(masked - m), 0.0)
    lse = jnp.log(jnp.sum(ex, axis=1, keepdims=True)) + m
    o_ref[...] = logits - lse


def _tc_call(body, out_shape, *args):
    return pl.pallas_call(body, out_shape=out_shape)(*args)


def kernel(x, edge_index, W0, b0, Wc, W1, b1):
    N, D = x.shape
    H = W0.shape[1]
    C = W1.shape[1]
    L = Wc.shape[0]
    E = edge_index.shape[1]
    NC, NS = _sc_info()

    src = edge_index[0].astype(jnp.int32)
    dst = edge_index[1].astype(jnp.int32)
    zeros = jnp.zeros((CHUNK, H), jnp.float32)

    f32 = jnp.float32
    h = _tc_call(_mm_relu_body, jax.ShapeDtypeStruct((N, H), f32),
                 x, W0, b0.reshape(1, H))
    x0 = h
    seg = _make_segment_sum(N, E, H)
    for l in range(L):
        beta = float(np.log(THETA / (l + 1) + 1.0))
        partials = seg(h, src, dst, zeros)
        h = _tc_call(functools.partial(_combine_body, beta=beta),
                     jax.ShapeDtypeStruct((N, H), f32),
                     partials[0], partials[1], x0, Wc[l])

    Wp = jnp.zeros((H, 128), f32).at[:, :C].set(W1)
    bp = jnp.zeros((1, 128), f32).at[0, :C].set(b1)
    out = _tc_call(functools.partial(_final_body, C=C),
                   jax.ShapeDtypeStruct((N, 128), f32),
                   h, Wp, bp)
    return out[:, :C]
